# Initial kernel scaffold; baseline (speedup 1.0000x reference)
import functools

import jax
import jax.numpy as jnp
from jax import lax
from jax.experimental import pallas as pl
from jax.experimental.pallas import tpu as pltpu
from jax.experimental.pallas import tpu_sc as plsc

N = 10000
E = 320000
D = 128

NC = 2
NS = 16
NW = NC * NS

CHUNK = 128
N_PAD = 10240
E_PAD = ((E + (CHUNK * NW) - 1) // (CHUNK * NW)) * (CHUNK * NW)
EPW = E_PAD // NW
NCH = EPW // CHUNK
NSL = N_PAD // NS


def _deg_body(col_ref, degp_ref, idx_v, ones_v, zer_v, deg_s):
    c = lax.axis_index("c")
    s = lax.axis_index("s")
    w = c * NS + s

    def fill(i, _):
        ones_v[pl.ds(i * 16, 16)] = jnp.ones((16,), jnp.float32)
        return _
    lax.fori_loop(0, CHUNK // 16, fill, None)

    def fillz(i, _):
        zer_v[pl.ds(i * 16, 16)] = jnp.zeros((16,), jnp.float32)
        return _
    lax.fori_loop(0, NSL // 16, fillz, None)

    pltpu.sync_copy(zer_v, deg_s.at[pl.ds(s * NSL, NSL)])

    pltpu.sync_copy(col_ref.at[pl.ds(w * NCH, NCH)], idx_v)
    plsc.subcore_barrier()

    def body(j, carry):
        pltpu.sync_copy(ones_v, deg_s.at[idx_v.at[j]], add=True)
        return carry
    lax.fori_loop(0, NCH, body, 0)
    plsc.subcore_barrier()

    pltpu.sync_copy(deg_s.at[pl.ds(s * NSL, NSL)],
                    degp_ref.at[c, pl.ds(s * NSL, NSL)])


def _msg_body(row_ref, col_ref, aux_ref, accp_ref,
              idx_r, idx_c, vals_v, tmp_v, zer_v, g_s, acc_s, sem):
    c = lax.axis_index("c")
    s = lax.axis_index("s")
    w = c * NS + s

    def fillz(i, _):
        zer_v[pl.ds(i * 16, 16)] = jnp.zeros((16,), jnp.float32)
        return _
    lax.fori_loop(0, NSL // 16, fillz, None)
    pltpu.sync_copy(zer_v, acc_s.at[pl.ds(s * NSL, NSL)])

    pltpu.sync_copy(aux_ref.at[0, pl.ds(s * NSL, NSL)], tmp_v)
    pltpu.sync_copy(tmp_v, g_s.at[pl.ds(s * NSL, NSL)])

    pltpu.sync_copy(row_ref.at[pl.ds(w * NCH, NCH)], idx_r)
    pltpu.sync_copy(col_ref.at[pl.ds(w * NCH, NCH)], idx_c)
    plsc.subcore_barrier()

    def body(j, carry):
        pltpu.async_copy(g_s.at[idx_r.at[j]], vals_v, sem).wait()
        pltpu.sync_copy(vals_v, acc_s.at[idx_c.at[j]], add=True)
        return carry
    lax.fori_loop(0, NCH, body, 0)
    plsc.subcore_barrier()

    pltpu.sync_copy(acc_s.at[pl.ds(s * NSL, NSL)],
                    accp_ref.at[c, pl.ds(s * NSL, NSL)])


def _sc_degree(col3):
    f = pl.kernel(
        _deg_body,
        out_type=jax.ShapeDtypeStruct((NC, N_PAD), jnp.float32),
        mesh=plsc.VectorSubcoreMesh(core_axis_name="c", subcore_axis_name="s"),
        scratch_types=[
            pltpu.VMEM((NCH, CHUNK), jnp.int32),
            pltpu.VMEM((CHUNK,), jnp.float32),
            pltpu.VMEM((NSL,), jnp.float32),
            pltpu.VMEM_SHARED((N_PAD,), jnp.float32),
        ],
    )
    return f(col3)


def _sc_message(row3, col3, aux):
    f = pl.kernel(
        _msg_body,
        out_type=jax.ShapeDtypeStruct((NC, N_PAD), jnp.float32),
        mesh=plsc.VectorSubcoreMesh(core_axis_name="c", subcore_axis_name="s"),
        scratch_types=[
            pltpu.VMEM((NCH, CHUNK), jnp.int32),
            pltpu.VMEM((NCH, CHUNK), jnp.int32),
            pltpu.VMEM((CHUNK,), jnp.float32),
            pltpu.VMEM((NSL,), jnp.float32),
            pltpu.VMEM((NSL,), jnp.float32),
            pltpu.VMEM_SHARED((N_PAD,), jnp.float32),
            pltpu.VMEM_SHARED((N_PAD,), jnp.float32),
            pltpu.SemaphoreType.DMA,
        ],
    )
    return f(row3, col3, aux)


ROWS_BLK = 2048
N_GRID = N_PAD // ROWS_BLK


def _prep_body(x_ref, w3t_ref, degp_ref, ab_ref, aux_ref):
    h = lax.dot_general(w3t_ref[...], x_ref[...],
                        (((1,), (1,)), ((), ())),
                        preferred_element_type=jnp.float32)
    deg = degp_ref[0:1, :] + degp_ref[1:2, :]
    alpha = ab_ref[0, 0]
    beta = ab_ref[0, 1]
    dis = lax.rsqrt(deg + 1.0)
    hm = h[0:1, :]
    hl = h[1:2, :]
    hg = h[2:3, :]
    g = hg * dis
    s1 = jax.nn.sigmoid(alpha * jnp.sqrt(deg) + beta)
    s2 = jax.nn.sigmoid(hm)
    selfterm = hg / (deg + 1.0)
    aux_ref[...] = jnp.concatenate([g, s1, s2, hl, dis, selfterm], axis=0)


def _tc_prep(x_pad, w3t, degp, ab):
    return pl.pallas_call(
        _prep_body,
        grid=(N_GRID,),
        in_specs=[
            pl.BlockSpec((ROWS_BLK, D), lambda i: (i, 0)),
            pl.BlockSpec((4, D), lambda i: (0, 0)),
            pl.BlockSpec((NC, ROWS_BLK), lambda i: (0, i)),
            pl.BlockSpec(memory_space=pltpu.SMEM),
        ],
        out_specs=pl.BlockSpec((6, ROWS_BLK), lambda i: (0, i)),
        out_shape=jax.ShapeDtypeStruct((6, N_PAD), jnp.float32),
    )(x_pad, w3t, degp, ab)


def _final_body(aux_ref, accp_ref, attn_ref, attnb_ref, fit_ref):
    acc = accp_ref[0:1, :] + accp_ref[1:2, :]
    dis = aux_ref[4:5, :]
    gcn = dis * acc + aux_ref[5:6, :]
    s3 = jax.nn.sigmoid(gcn + aux_ref[3:4, :])
    s1 = aux_ref[1:2, :]
    s2 = aux_ref[2:3, :]

    def logit(j):
        return (attn_ref[j, 0] * s1 + attn_ref[j, 1] * s2
                + attn_ref[j, 2] * s3 + attnb_ref[0, j])
    w0, w1, w2 = logit(0), logit(1), logit(2)
    m = jnp.maximum(jnp.maximum(w0, w1), w2)
    e0 = jnp.exp(w0 - m)
    e1 = jnp.exp(w1 - m)
    e2 = jnp.exp(w2 - m)
    z = e0 + e1 + e2
    fit_ref[...] = (e0 * s1 + e1 * s2 + e2 * s3) / z


def _tc_final(aux, accp, attn_w, attn_b2):
    return pl.pallas_call(
        _final_body,
        grid=(1,),
        in_specs=[
            pl.BlockSpec((6, N_PAD), lambda i: (0, 0)),
            pl.BlockSpec((NC, N_PAD), lambda i: (0, 0)),
            pl.BlockSpec(memory_space=pltpu.SMEM),
            pl.BlockSpec(memory_space=pltpu.SMEM),
        ],
        out_specs=pl.BlockSpec((1, N_PAD), lambda i: (0, 0)),
        out_shape=jax.ShapeDtypeStruct((1, N_PAD), jnp.float32),
    )(aux, accp, attn_w, attn_b2)


@jax.jit
def kernel(x, edge_index, alpha, beta, mlp_W, linear_W, gcn_W, attn_W, attn_b):
    row = edge_index[0]
    col = edge_index[1]
    pad = E_PAD - E
    pad_idx = N + (jnp.arange(pad, dtype=jnp.int32) % (N_PAD - N))
    row3 = jnp.concatenate([row, pad_idx]).reshape(E_PAD // CHUNK, CHUNK)
    col3 = jnp.concatenate([col, pad_idx]).reshape(E_PAD // CHUNK, CHUNK)

    x_pad = jnp.pad(x, ((0, N_PAD - N), (0, 0)))
    w3t = jnp.concatenate(
        [mlp_W, linear_W, gcn_W, jnp.zeros((D, 1), jnp.float32)], axis=1).T
    ab = jnp.stack([alpha, beta]).reshape(1, 2)
    attn_b2 = attn_b.reshape(1, 3)

    degp = _sc_degree(col3)
    aux = _tc_prep(x_pad, w3t, degp, ab)
    accp = _sc_message(row3, col3, aux)
    fit = _tc_final(aux, accp, attn_W, attn_b2)
    return fit[0, :N]


# speedup vs baseline: 100.4196x; 100.4196x over previous
import functools

import jax
import jax.numpy as jnp
from jax import lax
from jax.experimental import pallas as pl
from jax.experimental.pallas import tpu as pltpu
from jax.experimental.pallas import tpu_sc as plsc

N = 10000
E = 320000
D = 128

NC = 2
NS = 16
NW = NC * NS

CHUNK = 128
N_PAD = 10240
NCH = -(-E // (CHUNK * NW * 8)) * 8
E_PAD = NCH * CHUNK * NW
EPW = E_PAD // NW
NSL = N_PAD // NS


def _deg_body(col_ref, degp_ref, idx_v, ones_v, zer_v, deg_s):
    c = lax.axis_index("c")
    s = lax.axis_index("s")
    w = c * NS + s

    def fill(i, _):
        ones_v[pl.ds(i * 16, 16)] = jnp.ones((16,), jnp.float32)
        return _
    lax.fori_loop(0, CHUNK // 16, fill, None)

    def fillz(i, _):
        zer_v[pl.ds(i * 16, 16)] = jnp.zeros((16,), jnp.float32)
        return _
    lax.fori_loop(0, NSL // 16, fillz, None)

    pltpu.sync_copy(zer_v, deg_s.at[pl.ds(s * NSL, NSL)])

    pltpu.sync_copy(col_ref.at[pl.ds(w * NCH, NCH)], idx_v)
    plsc.subcore_barrier()

    def body(j, carry):
        pltpu.sync_copy(ones_v, deg_s.at[idx_v.at[j]], add=True)
        return carry
    lax.fori_loop(0, NCH, body, 0)
    plsc.subcore_barrier()

    pltpu.sync_copy(deg_s.at[pl.ds(s * NSL, NSL)],
                    degp_ref.at[pl.ds(c * N_PAD + s * NSL, NSL)])


def _msg_body(row_ref, col_ref, aux_ref, accp_ref,
              idx_r, idx_c, vals_v, tmp_v, zer_v, g_s, acc_s, sem):
    c = lax.axis_index("c")
    s = lax.axis_index("s")
    w = c * NS + s

    def fillz(i, _):
        zer_v[pl.ds(i * 16, 16)] = jnp.zeros((16,), jnp.float32)
        return _
    lax.fori_loop(0, NSL // 16, fillz, None)
    pltpu.sync_copy(zer_v, acc_s.at[pl.ds(s * NSL, NSL)])

    pltpu.sync_copy(aux_ref.at[0, pl.ds(s * NSL, NSL)], tmp_v)
    pltpu.sync_copy(tmp_v, g_s.at[pl.ds(s * NSL, NSL)])

    pltpu.sync_copy(row_ref.at[pl.ds(w * NCH, NCH)], idx_r)
    pltpu.sync_copy(col_ref.at[pl.ds(w * NCH, NCH)], idx_c)
    plsc.subcore_barrier()

    def body(j, carry):
        pltpu.async_copy(g_s.at[idx_r.at[j]], vals_v, sem).wait()
        pltpu.sync_copy(vals_v, acc_s.at[idx_c.at[j]], add=True)
        return carry
    lax.fori_loop(0, NCH, body, 0)
    plsc.subcore_barrier()

    pltpu.sync_copy(acc_s.at[pl.ds(s * NSL, NSL)],
                    accp_ref.at[pl.ds(c * N_PAD + s * NSL, NSL)])


def _sc_degree(col3):
    f = pl.kernel(
        _deg_body,
        out_type=jax.ShapeDtypeStruct((NC * N_PAD,), jnp.float32),
        mesh=plsc.VectorSubcoreMesh(core_axis_name="c", subcore_axis_name="s"),
        scratch_types=[
            pltpu.VMEM((NCH, CHUNK), jnp.int32),
            pltpu.VMEM((CHUNK,), jnp.float32),
            pltpu.VMEM((NSL,), jnp.float32),
            pltpu.VMEM_SHARED((N_PAD,), jnp.float32),
        ],
    )
    return f(col3)


def _sc_message(row3, col3, aux):
    f = pl.kernel(
        _msg_body,
        out_type=jax.ShapeDtypeStruct((NC * N_PAD,), jnp.float32),
        mesh=plsc.VectorSubcoreMesh(core_axis_name="c", subcore_axis_name="s"),
        scratch_types=[
            pltpu.VMEM((NCH, CHUNK), jnp.int32),
            pltpu.VMEM((NCH, CHUNK), jnp.int32),
            pltpu.VMEM((CHUNK,), jnp.float32),
            pltpu.VMEM((NSL,), jnp.float32),
            pltpu.VMEM((NSL,), jnp.float32),
            pltpu.VMEM_SHARED((N_PAD,), jnp.float32),
            pltpu.VMEM_SHARED((N_PAD,), jnp.float32),
            pltpu.SemaphoreType.DMA,
        ],
    )
    return f(row3, col3, aux)


ROWS_BLK = 2048
N_GRID = N_PAD // ROWS_BLK


def _prep_body(x_ref, w3t_ref, degp_ref, ab_ref, aux_ref):
    h = lax.dot_general(w3t_ref[...], x_ref[...],
                        (((1,), (1,)), ((), ())),
                        preferred_element_type=jnp.float32)
    deg = degp_ref[0:1, :] + degp_ref[1:2, :]
    alpha = ab_ref[0, 0]
    beta = ab_ref[0, 1]
    dis = lax.rsqrt(deg + 1.0)
    hm = h[0:1, :]
    hl = h[1:2, :]
    hg = h[2:3, :]
    g = hg * dis
    s1 = jax.nn.sigmoid(alpha * jnp.sqrt(deg) + beta)
    s2 = jax.nn.sigmoid(hm)
    selfterm = hg / (deg + 1.0)
    aux_ref[...] = jnp.concatenate([g, s1, s2, hl, dis, selfterm], axis=0)


def _tc_prep(x_pad, w3t, degp, ab):
    return pl.pallas_call(
        _prep_body,
        grid=(N_GRID,),
        in_specs=[
            pl.BlockSpec((ROWS_BLK, D), lambda i: (i, 0)),
            pl.BlockSpec((4, D), lambda i: (0, 0)),
            pl.BlockSpec((NC, ROWS_BLK), lambda i: (0, i)),
            pl.BlockSpec(memory_space=pltpu.SMEM),
        ],
        out_specs=pl.BlockSpec((6, ROWS_BLK), lambda i: (0, i)),
        out_shape=jax.ShapeDtypeStruct((6, N_PAD), jnp.float32),
    )(x_pad, w3t, degp, ab)


def _final_body(aux_ref, accp_ref, attn_ref, attnb_ref, fit_ref):
    acc = accp_ref[0:1, :] + accp_ref[1:2, :]
    dis = aux_ref[4:5, :]
    gcn = dis * acc + aux_ref[5:6, :]
    s3 = jax.nn.sigmoid(gcn + aux_ref[3:4, :])
    s1 = aux_ref[1:2, :]
    s2 = aux_ref[2:3, :]

    def logit(j):
        return (attn_ref[j, 0] * s1 + attn_ref[j, 1] * s2
                + attn_ref[j, 2] * s3 + attnb_ref[0, j])
    w0, w1, w2 = logit(0), logit(1), logit(2)
    m = jnp.maximum(jnp.maximum(w0, w1), w2)
    e0 = jnp.exp(w0 - m)
    e1 = jnp.exp(w1 - m)
    e2 = jnp.exp(w2 - m)
    z = e0 + e1 + e2
    fit_ref[...] = (e0 * s1 + e1 * s2 + e2 * s3) / z


def _tc_final(aux, accp, attn_w, attn_b2):
    return pl.pallas_call(
        _final_body,
        grid=(1,),
        in_specs=[
            pl.BlockSpec((6, N_PAD), lambda i: (0, 0)),
            pl.BlockSpec((NC, N_PAD), lambda i: (0, 0)),
            pl.BlockSpec(memory_space=pltpu.SMEM),
            pl.BlockSpec(memory_space=pltpu.SMEM),
        ],
        out_specs=pl.BlockSpec((1, N_PAD), lambda i: (0, 0)),
        out_shape=jax.ShapeDtypeStruct((1, N_PAD), jnp.float32),
    )(aux, accp, attn_w, attn_b2)


@jax.jit
def kernel(x, edge_index, alpha, beta, mlp_W, linear_W, gcn_W, attn_W, attn_b):
    row = edge_index[0]
    col = edge_index[1]
    pad = E_PAD - E
    pad_idx = N + (jnp.arange(pad, dtype=jnp.int32) % (N_PAD - N))
    row3 = jnp.concatenate([row, pad_idx]).reshape(E_PAD // CHUNK, CHUNK)
    col3 = jnp.concatenate([col, pad_idx]).reshape(E_PAD // CHUNK, CHUNK)

    x_pad = jnp.pad(x, ((0, N_PAD - N), (0, 0)))
    w3t = jnp.concatenate(
        [mlp_W, linear_W, gcn_W, jnp.zeros((D, 1), jnp.float32)], axis=1).T
    ab = jnp.stack([alpha, beta]).reshape(1, 2)
    attn_b2 = attn_b.reshape(1, 3)

    degp = _sc_degree(col3).reshape(NC, N_PAD)
    aux = _tc_prep(x_pad, w3t, degp, ab)
    accp = _sc_message(row3, col3, aux).reshape(NC, N_PAD)
    fit = _tc_final(aux, accp, attn_W, attn_b2)
    return fit[0, :N]


# speedup vs baseline: 123.2569x; 1.2274x over previous
import functools

import jax
import jax.numpy as jnp
from jax import lax
from jax.experimental import pallas as pl
from jax.experimental.pallas import tpu as pltpu
from jax.experimental.pallas import tpu_sc as plsc

N = 10000
E = 320000
D = 128

NC = 2
NS = 16
NW = NC * NS

CHUNK = 128
N_PAD = 10240
NCH = -(-E // (CHUNK * NW * 8)) * 8
E_PAD = NCH * CHUNK * NW
EPW = E_PAD // NW
NSL = N_PAD // NS


def _deg_body(ei_ref, degp_ref, idx_v, ones_v, zer_v, deg_s, sem):
    c = lax.axis_index("c")
    s = lax.axis_index("s")
    w = c * NS + s

    def fill(i, _):
        ones_v[pl.ds(i * 16, 16)] = jnp.ones((16,), jnp.float32)
        return _
    lax.fori_loop(0, CHUNK // 16, fill, None)

    def fillz(i, _):
        zer_v[pl.ds(i * 16, 16)] = jnp.zeros((16,), jnp.float32)
        return _
    lax.fori_loop(0, NSL // 16, fillz, None)

    pltpu.sync_copy(zer_v, deg_s.at[pl.ds(s * NSL, NSL)])

    pltpu.sync_copy(ei_ref.at[1, pl.ds(w * NCH, NCH)], idx_v)
    plsc.subcore_barrier()

    def body(j, carry):
        pltpu.async_copy(ones_v, deg_s.at[idx_v.at[j]], sem, add=True)
        return carry
    lax.fori_loop(0, NCH, body, 0)

    def drain(j, carry):
        pltpu.make_async_copy(ei_ref.at[1, pl.ds(0, 1)], idx_v.at[0:1], sem).wait()
        return carry
    lax.fori_loop(0, NCH, drain, 0)
    plsc.subcore_barrier()

    pltpu.sync_copy(deg_s.at[pl.ds(s * NSL, NSL)],
                    degp_ref.at[pl.ds(c * N_PAD + s * NSL, NSL)])


def _msg_body(ei_ref, aux_ref, accp_ref,
              idx_r, idx_c, vals_v, g_v, zer_v, acc_s, sem):
    c = lax.axis_index("c")
    s = lax.axis_index("s")
    w = c * NS + s

    def fillz(i, _):
        zer_v[pl.ds(i * 16, 16)] = jnp.zeros((16,), jnp.float32)
        return _
    lax.fori_loop(0, NSL // 16, fillz, None)
    pltpu.sync_copy(zer_v, acc_s.at[pl.ds(s * NSL, NSL)])

    pltpu.sync_copy(aux_ref.at[0, :], g_v)

    pltpu.sync_copy(ei_ref.at[0, pl.ds(w * NCH, NCH)], idx_r)
    pltpu.sync_copy(ei_ref.at[1, pl.ds(w * NCH, NCH)], idx_c)
    plsc.subcore_barrier()

    def body(j, carry):
        for cc in range(CHUNK // 16):
            i16 = idx_r[j, pl.ds(cc * 16, 16)]
            vals_v[j, pl.ds(cc * 16, 16)] = plsc.load_gather(g_v, [i16])
        pltpu.async_copy(vals_v.at[j], acc_s.at[idx_c.at[j]], sem, add=True)
        return carry
    lax.fori_loop(0, NCH, body, 0)

    def drain(j, carry):
        pltpu.make_async_copy(aux_ref.at[0, pl.ds(0, CHUNK)], vals_v.at[0], sem).wait()
        return carry
    lax.fori_loop(0, NCH, drain, 0)
    plsc.subcore_barrier()

    pltpu.sync_copy(acc_s.at[pl.ds(s * NSL, NSL)],
                    accp_ref.at[pl.ds(c * N_PAD + s * NSL, NSL)])


def _sc_degree(ei3):
    f = pl.kernel(
        _deg_body,
        out_type=jax.ShapeDtypeStruct((NC * N_PAD,), jnp.float32),
        mesh=plsc.VectorSubcoreMesh(core_axis_name="c", subcore_axis_name="s"),
        scratch_types=[
            pltpu.VMEM((NCH, CHUNK), jnp.int32),
            pltpu.VMEM((CHUNK,), jnp.float32),
            pltpu.VMEM((NSL,), jnp.float32),
            pltpu.VMEM_SHARED((N_PAD,), jnp.float32),
            pltpu.SemaphoreType.DMA,
        ],
    )
    return f(ei3)


def _sc_message(ei3, aux):
    f = pl.kernel(
        _msg_body,
        out_type=jax.ShapeDtypeStruct((NC * N_PAD,), jnp.float32),
        mesh=plsc.VectorSubcoreMesh(core_axis_name="c", subcore_axis_name="s"),
        compiler_params=pltpu.CompilerParams(needs_layout_passes=False),
        scratch_types=[
            pltpu.VMEM((NCH, CHUNK), jnp.int32),
            pltpu.VMEM((NCH, CHUNK), jnp.int32),
            pltpu.VMEM((NCH, CHUNK), jnp.float32),
            pltpu.VMEM((N_PAD,), jnp.float32),
            pltpu.VMEM((NSL,), jnp.float32),
            pltpu.VMEM_SHARED((N_PAD,), jnp.float32),
            pltpu.SemaphoreType.DMA,
        ],
    )
    return f(ei3, aux)


ROWS_BLK = 2048
N_GRID = N_PAD // ROWS_BLK


def _prep_body(x_ref, w3t_ref, degp_ref, ab_ref, aux_ref):
    h = lax.dot_general(w3t_ref[...], x_ref[...],
                        (((1,), (1,)), ((), ())),
                        preferred_element_type=jnp.float32)
    deg = degp_ref[0:1, :] + degp_ref[1:2, :]
    alpha = ab_ref[0, 0]
    beta = ab_ref[0, 1]
    dis = lax.rsqrt(deg + 1.0)
    hm = h[0:1, :]
    hl = h[1:2, :]
    hg = h[2:3, :]
    g = hg * dis
    s1 = jax.nn.sigmoid(alpha * jnp.sqrt(deg) + beta)
    s2 = jax.nn.sigmoid(hm)
    selfterm = hg / (deg + 1.0)
    aux_ref[...] = jnp.concatenate([g, s1, s2, hl, dis, selfterm], axis=0)


def _tc_prep(x_pad, w3t, degp, ab):
    return pl.pallas_call(
        _prep_body,
        grid=(N_GRID,),
        in_specs=[
            pl.BlockSpec((ROWS_BLK, D), lambda i: (i, 0)),
            pl.BlockSpec((4, D), lambda i: (0, 0)),
            pl.BlockSpec((NC, ROWS_BLK), lambda i: (0, i)),
            pl.BlockSpec(memory_space=pltpu.SMEM),
        ],
        out_specs=pl.BlockSpec((6, ROWS_BLK), lambda i: (0, i)),
        out_shape=jax.ShapeDtypeStruct((6, N_PAD), jnp.float32),
    )(x_pad, w3t, degp, ab)


def _final_body(aux_ref, accp_ref, attn_ref, attnb_ref, fit_ref):
    acc = accp_ref[0:1, :] + accp_ref[1:2, :]
    dis = aux_ref[4:5, :]
    gcn = dis * acc + aux_ref[5:6, :]
    s3 = jax.nn.sigmoid(gcn + aux_ref[3:4, :])
    s1 = aux_ref[1:2, :]
    s2 = aux_ref[2:3, :]

    def logit(j):
        return (attn_ref[j, 0] * s1 + attn_ref[j, 1] * s2
                + attn_ref[j, 2] * s3 + attnb_ref[0, j])
    w0, w1, w2 = logit(0), logit(1), logit(2)
    m = jnp.maximum(jnp.maximum(w0, w1), w2)
    e0 = jnp.exp(w0 - m)
    e1 = jnp.exp(w1 - m)
    e2 = jnp.exp(w2 - m)
    z = e0 + e1 + e2
    fit_ref[...] = (e0 * s1 + e1 * s2 + e2 * s3) / z


def _tc_final(aux, accp, attn_w, attn_b2):
    return pl.pallas_call(
        _final_body,
        grid=(1,),
        in_specs=[
            pl.BlockSpec((6, N_PAD), lambda i: (0, 0)),
            pl.BlockSpec((NC, N_PAD), lambda i: (0, 0)),
            pl.BlockSpec(memory_space=pltpu.SMEM),
            pl.BlockSpec(memory_space=pltpu.SMEM),
        ],
        out_specs=pl.BlockSpec((1, N_PAD), lambda i: (0, 0)),
        out_shape=jax.ShapeDtypeStruct((1, N_PAD), jnp.float32),
    )(aux, accp, attn_w, attn_b2)


@jax.jit
def kernel(x, edge_index, alpha, beta, mlp_W, linear_W, gcn_W, attn_W, attn_b):
    ei3 = jnp.pad(edge_index, ((0, 0), (0, E_PAD - E)),
                  constant_values=N).reshape(2, E_PAD // CHUNK, CHUNK)

    x_pad = jnp.pad(x, ((0, N_PAD - N), (0, 0)))
    w3t = jnp.concatenate(
        [mlp_W, linear_W, gcn_W, jnp.zeros((D, 1), jnp.float32)], axis=1).T
    ab = jnp.stack([alpha, beta]).reshape(1, 2)
    attn_b2 = attn_b.reshape(1, 3)

    degp = _sc_degree(ei3).reshape(NC, N_PAD)
    aux = _tc_prep(x_pad, w3t, degp, ab)
    accp = _sc_message(ei3, aux).reshape(NC, N_PAD)
    fit = _tc_final(aux, accp, attn_W, attn_b2)
    return fit[0, :N]


# speedup vs baseline: 125.8497x; 1.0210x over previous
import functools

import jax
import jax.numpy as jnp
from jax import lax
from jax.experimental import pallas as pl
from jax.experimental.pallas import tpu as pltpu
from jax.experimental.pallas import tpu_sc as plsc

N = 10000
E = 320000
D = 128

NC = 2
NS = 16
NW = NC * NS

CHUNK = 128
N_PAD = 10240
NCH = -(-E // (CHUNK * NW * 8)) * 8
E_PAD = NCH * CHUNK * NW
EPW = E_PAD // NW
NSL = N_PAD // NS


def _deg_body(ei_ref, degp_ref, idx_v, ones_v, zer_v, deg_s, sem):
    c = lax.axis_index("c")
    s = lax.axis_index("s")
    w = c * NS + s

    def fill(i, _):
        ones_v[pl.ds(i * 16, 16)] = jnp.ones((16,), jnp.float32)
        return _
    lax.fori_loop(0, CHUNK // 16, fill, None)

    def fillz(i, _):
        zer_v[pl.ds(i * 16, 16)] = jnp.zeros((16,), jnp.float32)
        return _
    lax.fori_loop(0, NSL // 16, fillz, None)

    pltpu.sync_copy(zer_v, deg_s.at[pl.ds(s * NSL, NSL)])

    pltpu.sync_copy(ei_ref.at[1, pl.ds(w * NCH, NCH)], idx_v)
    plsc.subcore_barrier()

    def body(j, carry):
        pltpu.async_copy(ones_v, deg_s.at[idx_v.at[j]], sem, add=True)
        return carry
    lax.fori_loop(0, NCH, body, 0)

    def drain(j, carry):
        pltpu.make_async_copy(ones_v, deg_s.at[idx_v.at[j]], sem).wait()
        return carry
    lax.fori_loop(0, NCH, drain, 0)
    plsc.subcore_barrier()

    pltpu.sync_copy(deg_s.at[pl.ds(s * NSL, NSL)],
                    degp_ref.at[pl.ds(c * N_PAD + s * NSL, NSL)])


def _msg_body(ei_ref, aux_ref, accp_ref,
              idx_r, idx_c, vals_v, g_v, zer_v, acc_s, sem):
    c = lax.axis_index("c")
    s = lax.axis_index("s")
    w = c * NS + s

    def fillz(i, _):
        zer_v[pl.ds(i * 16, 16)] = jnp.zeros((16,), jnp.float32)
        return _
    lax.fori_loop(0, NSL // 16, fillz, None)
    pltpu.sync_copy(zer_v, acc_s.at[pl.ds(s * NSL, NSL)])

    def stg(k, carry):
        t = lax.rem(s + k, NS) * NSL
        pltpu.async_copy(aux_ref.at[0, pl.ds(t, NSL)], g_v.at[pl.ds(t, NSL)],
                         sem)
        return carry
    lax.fori_loop(0, NS, stg, 0)
    pltpu.async_copy(ei_ref.at[0, pl.ds(w * NCH, NCH)], idx_r, sem)
    pltpu.async_copy(ei_ref.at[1, pl.ds(w * NCH, NCH)], idx_c, sem)

    def stgd(k, carry):
        t = lax.rem(s + k, NS) * NSL
        pltpu.make_async_copy(aux_ref.at[0, pl.ds(t, NSL)],
                              g_v.at[pl.ds(t, NSL)], sem).wait()
        return carry
    lax.fori_loop(0, NS, stgd, 0)
    pltpu.make_async_copy(ei_ref.at[0, pl.ds(w * NCH, NCH)], idx_r, sem).wait()
    pltpu.make_async_copy(ei_ref.at[1, pl.ds(w * NCH, NCH)], idx_c, sem).wait()
    plsc.subcore_barrier()

    def body(j, carry):
        for cc in range(CHUNK // 16):
            i16 = idx_r[j, pl.ds(cc * 16, 16)]
            vals_v[j, pl.ds(cc * 16, 16)] = plsc.load_gather(g_v, [i16])
        pltpu.async_copy(vals_v.at[j], acc_s.at[idx_c.at[j]], sem, add=True)
        return carry
    lax.fori_loop(0, NCH, body, 0)

    def drain(j, carry):
        pltpu.make_async_copy(vals_v.at[j], acc_s.at[idx_c.at[j]], sem).wait()
        return carry
    lax.fori_loop(0, NCH, drain, 0)
    plsc.subcore_barrier()

    pltpu.sync_copy(acc_s.at[pl.ds(s * NSL, NSL)],
                    accp_ref.at[pl.ds(c * N_PAD + s * NSL, NSL)])


def _sc_degree(ei3):
    f = pl.kernel(
        _deg_body,
        out_type=jax.ShapeDtypeStruct((NC * N_PAD,), jnp.float32),
        mesh=plsc.VectorSubcoreMesh(core_axis_name="c", subcore_axis_name="s"),
        scratch_types=[
            pltpu.VMEM((NCH, CHUNK), jnp.int32),
            pltpu.VMEM((CHUNK,), jnp.float32),
            pltpu.VMEM((NSL,), jnp.float32),
            pltpu.VMEM_SHARED((N_PAD,), jnp.float32),
            pltpu.SemaphoreType.DMA,
        ],
    )
    return f(ei3)


def _sc_message(ei3, aux):
    f = pl.kernel(
        _msg_body,
        out_type=jax.ShapeDtypeStruct((NC * N_PAD,), jnp.float32),
        mesh=plsc.VectorSubcoreMesh(core_axis_name="c", subcore_axis_name="s"),
        compiler_params=pltpu.CompilerParams(needs_layout_passes=False),
        scratch_types=[
            pltpu.VMEM((NCH, CHUNK), jnp.int32),
            pltpu.VMEM((NCH, CHUNK), jnp.int32),
            pltpu.VMEM((NCH, CHUNK), jnp.float32),
            pltpu.VMEM((N_PAD,), jnp.float32),
            pltpu.VMEM((NSL,), jnp.float32),
            pltpu.VMEM_SHARED((N_PAD,), jnp.float32),
            pltpu.SemaphoreType.DMA,
        ],
    )
    return f(ei3, aux)


ROWS_BLK = 2048
N_GRID = N_PAD // ROWS_BLK


def _prep_body(x_ref, w3t_ref, degp_ref, ab_ref, aux_ref):
    h = lax.dot_general(w3t_ref[...], x_ref[...],
                        (((1,), (1,)), ((), ())),
                        preferred_element_type=jnp.float32)
    deg = degp_ref[0:1, :] + degp_ref[1:2, :]
    alpha = ab_ref[0, 0]
    beta = ab_ref[0, 1]
    dis = lax.rsqrt(deg + 1.0)
    hm = h[0:1, :]
    hl = h[1:2, :]
    hg = h[2:3, :]
    g = hg * dis
    s1 = jax.nn.sigmoid(alpha * jnp.sqrt(deg) + beta)
    s2 = jax.nn.sigmoid(hm)
    selfterm = hg / (deg + 1.0)
    aux_ref[...] = jnp.concatenate([g, s1, s2, hl, dis, selfterm], axis=0)


def _tc_prep(x_pad, w3t, degp, ab):
    return pl.pallas_call(
        _prep_body,
        grid=(N_GRID,),
        in_specs=[
            pl.BlockSpec((ROWS_BLK, D), lambda i: (i, 0)),
            pl.BlockSpec((4, D), lambda i: (0, 0)),
            pl.BlockSpec((NC, ROWS_BLK), lambda i: (0, i)),
            pl.BlockSpec(memory_space=pltpu.SMEM),
        ],
        out_specs=pl.BlockSpec((6, ROWS_BLK), lambda i: (0, i)),
        out_shape=jax.ShapeDtypeStruct((6, N_PAD), jnp.float32),
    )(x_pad, w3t, degp, ab)


def _final_body(aux_ref, accp_ref, attn_ref, attnb_ref, fit_ref):
    acc = accp_ref[0:1, :] + accp_ref[1:2, :]
    dis = aux_ref[4:5, :]
    gcn = dis * acc + aux_ref[5:6, :]
    s3 = jax.nn.sigmoid(gcn + aux_ref[3:4, :])
    s1 = aux_ref[1:2, :]
    s2 = aux_ref[2:3, :]

    def logit(j):
        return (attn_ref[j, 0] * s1 + attn_ref[j, 1] * s2
                + attn_ref[j, 2] * s3 + attnb_ref[0, j])
    w0, w1, w2 = logit(0), logit(1), logit(2)
    m = jnp.maximum(jnp.maximum(w0, w1), w2)
    e0 = jnp.exp(w0 - m)
    e1 = jnp.exp(w1 - m)
    e2 = jnp.exp(w2 - m)
    z = e0 + e1 + e2
    fit_ref[...] = (e0 * s1 + e1 * s2 + e2 * s3) / z


def _tc_final(aux, accp, attn_w, attn_b2):
    return pl.pallas_call(
        _final_body,
        grid=(1,),
        in_specs=[
            pl.BlockSpec((6, N_PAD), lambda i: (0, 0)),
            pl.BlockSpec((NC, N_PAD), lambda i: (0, 0)),
            pl.BlockSpec(memory_space=pltpu.SMEM),
            pl.BlockSpec(memory_space=pltpu.SMEM),
        ],
        out_specs=pl.BlockSpec((1, N_PAD), lambda i: (0, 0)),
        out_shape=jax.ShapeDtypeStruct((1, N_PAD), jnp.float32),
    )(aux, accp, attn_w, attn_b2)


@jax.jit
def kernel(x, edge_index, alpha, beta, mlp_W, linear_W, gcn_W, attn_W, attn_b):
    ei3 = jnp.pad(edge_index, ((0, 0), (0, E_PAD - E)),
                  constant_values=N).reshape(2, E_PAD // CHUNK, CHUNK)

    x_pad = jnp.pad(x, ((0, N_PAD - N), (0, 0)))
    w3t = jnp.concatenate(
        [mlp_W, linear_W, gcn_W, jnp.zeros((D, 1), jnp.float32)], axis=1).T
    ab = jnp.stack([alpha, beta]).reshape(1, 2)
    attn_b2 = attn_b.reshape(1, 3)

    degp = _sc_degree(ei3).reshape(NC, N_PAD)
    aux = _tc_prep(x_pad, w3t, degp, ab)
    accp = _sc_message(ei3, aux).reshape(NC, N_PAD)
    fit = _tc_final(aux, accp, attn_W, attn_b2)
    return fit[0, :N]


# speedup vs baseline: 144.7135x; 1.1499x over previous
import functools

import jax
import jax.numpy as jnp
from jax import lax
from jax.experimental import pallas as pl
from jax.experimental.pallas import tpu as pltpu
from jax.experimental.pallas import tpu_sc as plsc

N = 10000
E = 320000
D = 128

NC = 2
NS = 16
NW = NC * NS

CHUNK = 128
N_PAD = 10240
NCH = -(-E // (CHUNK * NW * 8)) * 8
E_PAD = NCH * CHUNK * NW
EPW = E_PAD // NW
NSL = N_PAD // NS


def _deg_body(ei_ref, deg0_ref, deg1_ref, idx_v, ones_v, zer_v, deg_s, sem):
    c = lax.axis_index("c")
    s = lax.axis_index("s")
    w = c * NS + s

    def fill(i, _):
        ones_v[pl.ds(i * 16, 16)] = jnp.ones((16,), jnp.float32)
        return _
    lax.fori_loop(0, CHUNK // 16, fill, None)

    def fillz(i, _):
        zer_v[pl.ds(i * 16, 16)] = jnp.zeros((16,), jnp.float32)
        return _
    lax.fori_loop(0, NSL // 16, fillz, None)

    pltpu.sync_copy(zer_v, deg_s.at[pl.ds(s * NSL, NSL)])

    pltpu.sync_copy(ei_ref.at[1, pl.ds(w * NCH, NCH)], idx_v)
    plsc.subcore_barrier()

    def body(j, carry):
        pltpu.async_copy(ones_v, deg_s.at[idx_v.at[j]], sem, add=True)
        return carry
    lax.fori_loop(0, NCH, body, 0)

    def drain(j, carry):
        pltpu.make_async_copy(ones_v, deg_s.at[idx_v.at[j]], sem).wait()
        return carry
    lax.fori_loop(0, NCH, drain, 0)
    plsc.subcore_barrier()

    @pl.when(c == 0)
    def _():
        pltpu.sync_copy(deg_s.at[pl.ds(s * NSL, NSL)],
                        deg0_ref.at[0, pl.ds(s * NSL, NSL)])

    @pl.when(c == 1)
    def _():
        pltpu.sync_copy(deg_s.at[pl.ds(s * NSL, NSL)],
                        deg1_ref.at[0, pl.ds(s * NSL, NSL)])


def _msg_body(ei_ref, aux_ref, acc0_ref, acc1_ref,
              idx_r, idx_c, vals_v, g_v, zer_v, acc_s, sem):
    c = lax.axis_index("c")
    s = lax.axis_index("s")
    w = c * NS + s

    def fillz(i, _):
        zer_v[pl.ds(i * 16, 16)] = jnp.zeros((16,), jnp.float32)
        return _
    lax.fori_loop(0, NSL // 16, fillz, None)
    pltpu.sync_copy(zer_v, acc_s.at[pl.ds(s * NSL, NSL)])

    def stg(k, carry):
        t = lax.rem(s + k, NS) * NSL
        pltpu.async_copy(aux_ref.at[0, pl.ds(t, NSL)], g_v.at[pl.ds(t, NSL)],
                         sem)
        return carry
    lax.fori_loop(0, NS, stg, 0)
    pltpu.async_copy(ei_ref.at[0, pl.ds(w * NCH, NCH)], idx_r, sem)
    pltpu.async_copy(ei_ref.at[1, pl.ds(w * NCH, NCH)], idx_c, sem)

    def stgd(k, carry):
        t = lax.rem(s + k, NS) * NSL
        pltpu.make_async_copy(aux_ref.at[0, pl.ds(t, NSL)],
                              g_v.at[pl.ds(t, NSL)], sem).wait()
        return carry
    lax.fori_loop(0, NS, stgd, 0)
    pltpu.make_async_copy(ei_ref.at[0, pl.ds(w * NCH, NCH)], idx_r, sem).wait()
    pltpu.make_async_copy(ei_ref.at[1, pl.ds(w * NCH, NCH)], idx_c, sem).wait()
    plsc.subcore_barrier()

    def body(j, carry):
        for cc in range(CHUNK // 16):
            i16 = idx_r[j, pl.ds(cc * 16, 16)]
            vals_v[j, pl.ds(cc * 16, 16)] = plsc.load_gather(g_v, [i16])
        pltpu.async_copy(vals_v.at[j], acc_s.at[idx_c.at[j]], sem, add=True)
        return carry
    lax.fori_loop(0, NCH, body, 0)

    def drain(j, carry):
        pltpu.make_async_copy(vals_v.at[j], acc_s.at[idx_c.at[j]], sem).wait()
        return carry
    lax.fori_loop(0, NCH, drain, 0)
    plsc.subcore_barrier()

    @pl.when(c == 0)
    def _():
        pltpu.sync_copy(acc_s.at[pl.ds(s * NSL, NSL)],
                        acc0_ref.at[0, pl.ds(s * NSL, NSL)])

    @pl.when(c == 1)
    def _():
        pltpu.sync_copy(acc_s.at[pl.ds(s * NSL, NSL)],
                        acc1_ref.at[0, pl.ds(s * NSL, NSL)])


def _sc_degree(ei3):
    f = pl.kernel(
        _deg_body,
        out_type=(jax.ShapeDtypeStruct((1, N_PAD), jnp.float32),
                  jax.ShapeDtypeStruct((1, N_PAD), jnp.float32)),
        mesh=plsc.VectorSubcoreMesh(core_axis_name="c", subcore_axis_name="s"),
        scratch_types=[
            pltpu.VMEM((NCH, CHUNK), jnp.int32),
            pltpu.VMEM((CHUNK,), jnp.float32),
            pltpu.VMEM((NSL,), jnp.float32),
            pltpu.VMEM_SHARED((N_PAD,), jnp.float32),
            pltpu.SemaphoreType.DMA,
        ],
    )
    return f(ei3)


def _sc_message(ei3, aux):
    f = pl.kernel(
        _msg_body,
        out_type=(jax.ShapeDtypeStruct((1, N_PAD), jnp.float32),
                  jax.ShapeDtypeStruct((1, N_PAD), jnp.float32)),
        mesh=plsc.VectorSubcoreMesh(core_axis_name="c", subcore_axis_name="s"),
        compiler_params=pltpu.CompilerParams(needs_layout_passes=False),
        scratch_types=[
            pltpu.VMEM((NCH, CHUNK), jnp.int32),
            pltpu.VMEM((NCH, CHUNK), jnp.int32),
            pltpu.VMEM((NCH, CHUNK), jnp.float32),
            pltpu.VMEM((N_PAD,), jnp.float32),
            pltpu.VMEM((NSL,), jnp.float32),
            pltpu.VMEM_SHARED((N_PAD,), jnp.float32),
            pltpu.SemaphoreType.DMA,
        ],
    )
    return f(ei3, aux)


ROWS_BLK = 2048
N_GRID = N_PAD // ROWS_BLK


def _matvec_body(x_ref, w3t_ref, h_ref):
    h_ref[...] = lax.dot_general(w3t_ref[...], x_ref[...],
                                 (((1,), (1,)), ((), ())),
                                 preferred_element_type=jnp.float32)


def _tc_matvec(x, w3t):
    return pl.pallas_call(
        _matvec_body,
        grid=(N_GRID,),
        in_specs=[
            pl.BlockSpec((ROWS_BLK, D), lambda i: (i, 0)),
            pl.BlockSpec((4, D), lambda i: (0, 0)),
        ],
        out_specs=pl.BlockSpec((4, ROWS_BLK), lambda i: (0, i)),
        out_shape=jax.ShapeDtypeStruct((4, N_PAD), jnp.float32),
    )(x, w3t)


def _prep_body(h_ref, deg0_ref, deg1_ref, ab_ref, aux_ref):
    deg = deg0_ref[...] + deg1_ref[...]
    alpha = ab_ref[0, 0]
    beta = ab_ref[0, 1]
    dis = lax.rsqrt(deg + 1.0)
    hm = h_ref[0:1, :]
    hl = h_ref[1:2, :]
    hg = h_ref[2:3, :]
    g = hg * dis
    s1 = jax.nn.sigmoid(alpha * jnp.sqrt(deg) + beta)
    s2 = jax.nn.sigmoid(hm)
    selfterm = hg / (deg + 1.0)
    aux_ref[...] = jnp.concatenate([g, s1, s2, hl, dis, selfterm], axis=0)


def _tc_prep(h, deg0, deg1, ab):
    return pl.pallas_call(
        _prep_body,
        in_specs=[
            pl.BlockSpec((4, N_PAD), lambda: (0, 0)),
            pl.BlockSpec((1, N_PAD), lambda: (0, 0)),
            pl.BlockSpec((1, N_PAD), lambda: (0, 0)),
            pl.BlockSpec(memory_space=pltpu.SMEM),
        ],
        out_specs=pl.BlockSpec((6, N_PAD), lambda: (0, 0)),
        out_shape=jax.ShapeDtypeStruct((6, N_PAD), jnp.float32),
    )(h, deg0, deg1, ab)


def _final_body(aux_ref, acc0_ref, acc1_ref, attn_ref, attnb_ref, fit_ref):
    acc = acc0_ref[...] + acc1_ref[...]
    dis = aux_ref[4:5, :]
    gcn = dis * acc + aux_ref[5:6, :]
    s3 = jax.nn.sigmoid(gcn + aux_ref[3:4, :])
    s1 = aux_ref[1:2, :]
    s2 = aux_ref[2:3, :]

    def logit(j):
        return (attn_ref[j, 0] * s1 + attn_ref[j, 1] * s2
                + attn_ref[j, 2] * s3 + attnb_ref[0, j])
    w0, w1, w2 = logit(0), logit(1), logit(2)
    m = jnp.maximum(jnp.maximum(w0, w1), w2)
    e0 = jnp.exp(w0 - m)
    e1 = jnp.exp(w1 - m)
    e2 = jnp.exp(w2 - m)
    z = e0 + e1 + e2
    fit_ref[...] = (e0 * s1 + e1 * s2 + e2 * s3) / z


def _tc_final(aux, acc0, acc1, attn_w, attn_b2):
    return pl.pallas_call(
        _final_body,
        in_specs=[
            pl.BlockSpec((6, N_PAD), lambda: (0, 0)),
            pl.BlockSpec((1, N_PAD), lambda: (0, 0)),
            pl.BlockSpec((1, N_PAD), lambda: (0, 0)),
            pl.BlockSpec(memory_space=pltpu.SMEM),
            pl.BlockSpec(memory_space=pltpu.SMEM),
        ],
        out_specs=pl.BlockSpec((1, N_PAD), lambda: (0, 0)),
        out_shape=jax.ShapeDtypeStruct((1, N_PAD), jnp.float32),
    )(aux, acc0, acc1, attn_w, attn_b2)


@jax.jit
def kernel(x, edge_index, alpha, beta, mlp_W, linear_W, gcn_W, attn_W, attn_b):
    ei3 = jnp.pad(edge_index, ((0, 0), (0, E_PAD - E)),
                  constant_values=N).reshape(2, E_PAD // CHUNK, CHUNK)

    w3t = jnp.concatenate(
        [mlp_W, linear_W, gcn_W, jnp.zeros((D, 1), jnp.float32)], axis=1).T
    ab = jnp.stack([alpha, beta]).reshape(1, 2)
    attn_b2 = attn_b.reshape(1, 3)

    deg0, deg1 = _sc_degree(ei3)
    h = _tc_matvec(x, w3t)
    aux = _tc_prep(h, deg0, deg1, ab)
    acc0, acc1 = _sc_message(ei3, aux)
    fit = _tc_final(aux, acc0, acc1, attn_W, attn_b2)
    return fit[0, :N]
